# baseline (speedup 1.0000x reference)
import functools
import jax
import jax.numpy as jnp
from jax import lax
from jax.experimental import pallas as pl
from jax.experimental.pallas import tpu as pltpu
from jax.experimental.pallas import tpu_sc as plsc

NC = 2
NS = 16
NW = NC * NS
L = 16

PACK_SHIFT = 14
PACK_MASK = (1 << PACK_SHIFT) - 1


def _wid():
    return lax.axis_index("s") * NC + lax.axis_index("c")


def _mesh():
    return plsc.VectorSubcoreMesh(
        core_axis_name="c", subcore_axis_name="s", num_cores=NC, num_subcores=NS
    )


def _deg_pack_kernel(E, N):
    Ew = E // NW
    assert E % (NW * L) == 0 and N % L == 0

    def body(src_hbm, dst_hbm, deg_hbm, packed_hbm, src_v, dst_v, pk_v, acc_v):
        w = _wid()
        base = w * Ew
        pltpu.sync_copy(src_hbm.at[pl.ds(base, Ew)], src_v)
        pltpu.sync_copy(dst_hbm.at[pl.ds(base, Ew)], dst_v)

        def zero(i, c):
            acc_v[pl.ds(i * L, L)] = jnp.zeros((L,), jnp.float32)
            return c

        lax.fori_loop(0, N // L, zero, 0, unroll=8)

        ones = jnp.ones((L,), jnp.float32)

        @plsc.parallel_loop(0, Ew, step=L, unroll=8)
        def _(i):
            s16 = src_v[pl.ds(i, L)]
            d16 = dst_v[pl.ds(i, L)]
            pk_v[pl.ds(i, L)] = jnp.bitwise_or(
                jnp.left_shift(s16, PACK_SHIFT), d16
            )
            plsc.addupdate_scatter(acc_v, [d16], ones)
        pltpu.sync_copy(pk_v, packed_hbm.at[pl.ds(base, Ew)])
        pltpu.sync_copy(acc_v, deg_hbm.at[pl.ds(w * N, N)])

    return pl.kernel(
        body,
        out_type=(
            jax.ShapeDtypeStruct((NW * N,), jnp.float32),
            jax.ShapeDtypeStruct((E,), jnp.int32),
        ),
        mesh=_mesh(),
        compiler_params=pltpu.CompilerParams(needs_layout_passes=False),
        scratch_types=[
            pltpu.VMEM((Ew,), jnp.int32),
            pltpu.VMEM((Ew,), jnp.int32),
            pltpu.VMEM((Ew,), jnp.int32),
            pltpu.VMEM((N,), jnp.float32),
        ],
    )


NF = 4
NP = 4


def _agg_kernel(E, N, C=8000):
    Ep = E // NP
    while Ep % C or C % L:
        C //= 2
    nchunks = Ep // C
    assert Ep % C == 0 and C % L == 0 and N % L == 0

    def body(m_hbm, packed_hbm, out_hbm, mv, av, pk0, pk1, sem0, sem1, semm):
        c = lax.axis_index("c")
        s = lax.axis_index("s")
        g = lax.rem(s, 8)
        q = c * 2 + lax.div(s, 8)
        f0 = NF * g
        ebase = q * Ep
        bufs = (pk0, pk1)
        sems = (sem0, sem1)
        mcps = [
            pltpu.async_copy(m_hbm.at[pl.ds((f0 + j) * N, N)], mv[j], semm)
            for j in range(NF)
        ]
        handles = {0: pltpu.async_copy(packed_hbm.at[pl.ds(ebase, C)], pk0, sem0)}

        def zero(i, cc):
            for j in range(NF):
                av[j][pl.ds(i * L, L)] = jnp.zeros((L,), jnp.float32)
            return cc

        lax.fori_loop(0, N // L, zero, 0, unroll=8)
        for h in mcps:
            h.wait()

        for ci in range(nchunks):
            b = ci % 2
            if ci + 1 < nchunks:
                handles[ci + 1] = pltpu.async_copy(
                    packed_hbm.at[pl.ds(ebase + (ci + 1) * C, C)],
                    bufs[(ci + 1) % 2],
                    sems[(ci + 1) % 2],
                )
            handles[ci].wait()
            pk_v = bufs[b]

            @plsc.parallel_loop(0, C, step=L, unroll=8)
            def _(i):
                pk = pk_v[pl.ds(i, L)]
                s16 = lax.shift_right_logical(pk, PACK_SHIFT)
                d16 = jnp.bitwise_and(pk, PACK_MASK)
                for j in range(NF):
                    plsc.addupdate_scatter(
                        av[j], [d16], plsc.load_gather(mv[j], [s16])
                    )

        obase = q * NW * N + f0 * N
        for j in range(NF):
            pltpu.sync_copy(av[j], out_hbm.at[pl.ds(obase + j * N, N)])

    def body_wrap(m_hbm, packed_hbm, out_hbm, m0, m1, m2, m3,
                  a0, a1, a2, a3, pk0, pk1, sem0, sem1, semm):
        return body(m_hbm, packed_hbm, out_hbm, (m0, m1, m2, m3),
                    (a0, a1, a2, a3), pk0, pk1, sem0, sem1, semm)

    return pl.kernel(
        body_wrap,
        out_type=jax.ShapeDtypeStruct((NP * NW * N,), jnp.float32),
        mesh=_mesh(),
        compiler_params=pltpu.CompilerParams(needs_layout_passes=False),
        scratch_types=[pltpu.VMEM((N,), jnp.float32)] * 8
        + [
            pltpu.VMEM((C,), jnp.int32),
            pltpu.VMEM((C,), jnp.int32),
            pltpu.SemaphoreType.DMA,
            pltpu.SemaphoreType.DMA,
            pltpu.SemaphoreType.DMA,
        ],
    )


def _pre_body(degp_ref, x_ref, w1t_ref, m_ref, dis_ref):
    deg = jnp.sum(degp_ref[...], axis=0, keepdims=True) + 1.0
    dis = lax.rsqrt(deg)
    h = lax.dot_general(
        w1t_ref[...], x_ref[...], (((1,), (1,)), ((), ())),
        preferred_element_type=jnp.float32,
    )
    m_ref[...] = h * dis
    dis_ref[...] = dis


def _mid_body(agg_ref, m_ref, dis_ref, b_ref, g_ref, be_ref, wnt_ref, out_ref):
    dis = dis_ref[...]
    a = dis * (agg_ref[0] + agg_ref[1] + agg_ref[2] + agg_ref[3]
               + m_ref[...]) + b_ref[...]
    a = jnp.maximum(a, 0.0)
    mu = jnp.mean(a, axis=0, keepdims=True)
    var = jnp.mean((a - mu) * (a - mu), axis=0, keepdims=True)
    a = (a - mu) * lax.rsqrt(var + 1e-5) * g_ref[...] + be_ref[...]
    h = lax.dot_general(
        wnt_ref[...], a, (((1,), (0,)), ((), ())),
        preferred_element_type=jnp.float32,
    )
    out_ref[...] = h * dis


def _post_body(agg_ref, m_ref, dis_ref, b_ref, wp1t_ref, bp1_ref, wp2t_ref,
               bp2_ref, out_ref):
    dis = dis_ref[...]
    a = dis * (agg_ref[0] + agg_ref[1] + agg_ref[2] + agg_ref[3]
               + m_ref[...]) + b_ref[...]
    a = jnp.maximum(a, 0.0)
    p = lax.dot_general(
        wp1t_ref[...], a, (((1,), (0,)), ((), ())),
        preferred_element_type=jnp.float32,
    ) + bp1_ref[...]
    p = jnp.maximum(p, 0.0)
    o = lax.dot_general(
        wp2t_ref[...], p, (((1,), (0,)), ((), ())),
        preferred_element_type=jnp.float32,
    ) + bp2_ref[...]
    o = o - jnp.max(o, axis=0, keepdims=True)
    e = jnp.exp(o)
    out_ref[...] = e / jnp.sum(e, axis=0, keepdims=True)


@jax.jit
def kernel(x, edge_index, W1, b1, W2, b2, W3, b3, g1, be1, g2, be2,
           Wp1, bp1, Wp2, bp2):
    N, D = x.shape
    E = edge_index.shape[1]
    H = W1.shape[1]
    OUT = Wp2.shape[1]
    assert H == NW

    src = edge_index[0]
    dst = edge_index[1]

    deg_flat, packed = _deg_pack_kernel(E, N)(src, dst)
    degp = deg_flat.reshape(NW, N)

    agg_fn = _agg_kernel(E, N)

    m1, dis = pl.pallas_call(
        _pre_body,
        out_shape=(
            jax.ShapeDtypeStruct((H, N), jnp.float32),
            jax.ShapeDtypeStruct((1, N), jnp.float32),
        ),
    )(degp, x, W1.T)

    agg1 = agg_fn(m1.reshape(-1), packed).reshape(NP, H, N)

    mid = pl.pallas_call(
        _mid_body, out_shape=jax.ShapeDtypeStruct((H, N), jnp.float32)
    )

    m2 = mid(agg1, m1, dis, b1[:, None], g1[:, None], be1[:, None], W2.T)
    agg2 = agg_fn(m2.reshape(-1), packed).reshape(NP, H, N)

    m3 = mid(agg2, m2, dis, b2[:, None], g2[:, None], be2[:, None], W3.T)
    agg3 = agg_fn(m3.reshape(-1), packed).reshape(NP, H, N)

    out_t = pl.pallas_call(
        _post_body, out_shape=jax.ShapeDtypeStruct((OUT, N), jnp.float32)
    )(agg3, m3, dis, b3[:, None], Wp1.T, bp1[:, None], Wp2.T, bp2[:, None])

    return out_t.T


# speedup vs baseline: 1.0262x; 1.0262x over previous
import functools
import jax
import jax.numpy as jnp
from jax import lax
from jax.experimental import pallas as pl
from jax.experimental.pallas import tpu as pltpu
from jax.experimental.pallas import tpu_sc as plsc

NC = 2
NS = 16
NW = NC * NS
L = 16

PACK_SHIFT = 14
PACK_MASK = (1 << PACK_SHIFT) - 1


def _wid():
    return lax.axis_index("s") * NC + lax.axis_index("c")


def _mesh():
    return plsc.VectorSubcoreMesh(
        core_axis_name="c", subcore_axis_name="s", num_cores=NC, num_subcores=NS
    )


def _deg_pack_kernel(E, N):
    Ew = E // NW
    assert E % (NW * L) == 0 and N % L == 0

    def body(edge_hbm, deg_hbm, packed_hbm, src_v, dst_v, pk_v, acc_v):
        w = _wid()
        base = w * Ew
        pltpu.sync_copy(edge_hbm.at[pl.ds(base, Ew)], src_v)
        pltpu.sync_copy(edge_hbm.at[pl.ds(E + base, Ew)], dst_v)

        def zero(i, c):
            acc_v[pl.ds(i * L, L)] = jnp.zeros((L,), jnp.float32)
            return c

        lax.fori_loop(0, N // L, zero, 0, unroll=8)

        ones = jnp.ones((L,), jnp.float32)

        @plsc.parallel_loop(0, Ew, step=L, unroll=8)
        def _(i):
            s16 = src_v[pl.ds(i, L)]
            d16 = dst_v[pl.ds(i, L)]
            pk_v[pl.ds(i, L)] = jnp.bitwise_or(
                jnp.left_shift(s16, PACK_SHIFT), d16
            )
            plsc.addupdate_scatter(acc_v, [d16], ones)
        pltpu.sync_copy(pk_v, packed_hbm.at[pl.ds(base, Ew)])
        pltpu.sync_copy(acc_v, deg_hbm.at[pl.ds(w * N, N)])

    return pl.kernel(
        body,
        out_type=(
            jax.ShapeDtypeStruct((NW * N,), jnp.float32),
            jax.ShapeDtypeStruct((E,), jnp.int32),
        ),
        mesh=_mesh(),
        compiler_params=pltpu.CompilerParams(needs_layout_passes=False),
        scratch_types=[
            pltpu.VMEM((Ew,), jnp.int32),
            pltpu.VMEM((Ew,), jnp.int32),
            pltpu.VMEM((Ew,), jnp.int32),
            pltpu.VMEM((N,), jnp.float32),
        ],
    )


def _agg_kernel(E, N, C=16000):
    Eh = E // NC
    while Eh % C or C % L:
        C //= 2
    nchunks = Eh // C
    assert Eh % C == 0 and C % L == 0 and N % L == 0

    def body(m_hbm, packed_hbm, out_hbm, m0, m1, a0, a1, pk0, pk1,
             sem0, sem1, semm):
        c = lax.axis_index("c")
        s = lax.axis_index("s")
        f0 = 2 * s
        ebase = c * Eh
        bufs = (pk0, pk1)
        sems = (sem0, sem1)
        mcp0 = pltpu.async_copy(m_hbm.at[pl.ds(f0 * N, N)], m0, semm)
        mcp1 = pltpu.async_copy(m_hbm.at[pl.ds((f0 + 1) * N, N)], m1, semm)
        handles = {0: pltpu.async_copy(packed_hbm.at[pl.ds(ebase, C)], pk0, sem0)}

        def zero(i, cc):
            a0[pl.ds(i * L, L)] = jnp.zeros((L,), jnp.float32)
            a1[pl.ds(i * L, L)] = jnp.zeros((L,), jnp.float32)
            return cc

        lax.fori_loop(0, N // L, zero, 0, unroll=8)
        mcp0.wait()
        mcp1.wait()

        for ci in range(nchunks):
            b = ci % 2
            if ci + 1 < nchunks:
                handles[ci + 1] = pltpu.async_copy(
                    packed_hbm.at[pl.ds(ebase + (ci + 1) * C, C)],
                    bufs[(ci + 1) % 2],
                    sems[(ci + 1) % 2],
                )
            handles[ci].wait()
            pk_v = bufs[b]

            @plsc.parallel_loop(0, C, step=L, unroll=8)
            def _(i):
                pk = pk_v[pl.ds(i, L)]
                s16 = lax.shift_right_logical(pk, PACK_SHIFT)
                d16 = jnp.bitwise_and(pk, PACK_MASK)
                plsc.addupdate_scatter(a0, [d16], plsc.load_gather(m0, [s16]))
                plsc.addupdate_scatter(a1, [d16], plsc.load_gather(m1, [s16]))

        obase = c * NW * N + f0 * N
        pltpu.sync_copy(a0, out_hbm.at[pl.ds(obase, N)])
        pltpu.sync_copy(a1, out_hbm.at[pl.ds(obase + N, N)])

    return pl.kernel(
        body,
        out_type=jax.ShapeDtypeStruct((NC * NW * N,), jnp.float32),
        mesh=_mesh(),
        compiler_params=pltpu.CompilerParams(needs_layout_passes=False),
        scratch_types=[
            pltpu.VMEM((N,), jnp.float32),
            pltpu.VMEM((N,), jnp.float32),
            pltpu.VMEM((N,), jnp.float32),
            pltpu.VMEM((N,), jnp.float32),
            pltpu.VMEM((C,), jnp.int32),
            pltpu.VMEM((C,), jnp.int32),
            pltpu.SemaphoreType.DMA,
            pltpu.SemaphoreType.DMA,
            pltpu.SemaphoreType.DMA,
        ],
    )


def _pre_body(degp_ref, x_ref, w1t_ref, m_ref, dis_ref):
    deg = jnp.sum(degp_ref[...], axis=0, keepdims=True) + 1.0
    dis = lax.rsqrt(deg)
    h = lax.dot_general(
        w1t_ref[...], x_ref[...], (((1,), (1,)), ((), ())),
        preferred_element_type=jnp.float32,
    )
    m_ref[...] = h * dis
    dis_ref[...] = dis


def _mid_body(agg_ref, m_ref, dis_ref, b_ref, g_ref, be_ref, wnt_ref, out_ref):
    dis = dis_ref[...]
    a = dis * (agg_ref[0] + agg_ref[1] + m_ref[...]) + b_ref[...]
    a = jnp.maximum(a, 0.0)
    mu = jnp.mean(a, axis=0, keepdims=True)
    var = jnp.mean((a - mu) * (a - mu), axis=0, keepdims=True)
    a = (a - mu) * lax.rsqrt(var + 1e-5) * g_ref[...] + be_ref[...]
    h = lax.dot_general(
        wnt_ref[...], a, (((1,), (0,)), ((), ())),
        preferred_element_type=jnp.float32,
    )
    out_ref[...] = h * dis


def _post_body(agg_ref, m_ref, dis_ref, b_ref, wp1t_ref, bp1_ref, wp2_ref,
               bp2_ref, out_ref):
    dis = dis_ref[...]
    a = dis * (agg_ref[0] + agg_ref[1] + m_ref[...]) + b_ref[...]
    a = jnp.maximum(a, 0.0)
    p = lax.dot_general(
        wp1t_ref[...], a, (((1,), (0,)), ((), ())),
        preferred_element_type=jnp.float32,
    ) + bp1_ref[...]
    p = jnp.maximum(p, 0.0)
    o = lax.dot_general(
        p, wp2_ref[...], (((0,), (0,)), ((), ())),
        preferred_element_type=jnp.float32,
    ) + bp2_ref[...]
    o = o - jnp.max(o, axis=1, keepdims=True)
    e = jnp.exp(o)
    out_ref[...] = e / jnp.sum(e, axis=1, keepdims=True)


@jax.jit
def kernel(x, edge_index, W1, b1, W2, b2, W3, b3, g1, be1, g2, be2,
           Wp1, bp1, Wp2, bp2):
    N, D = x.shape
    E = edge_index.shape[1]
    H = W1.shape[1]
    OUT = Wp2.shape[1]
    assert H == NW

    deg_flat, packed = _deg_pack_kernel(E, N)(edge_index.reshape(-1))
    degp = deg_flat.reshape(NW, N)

    agg_fn = _agg_kernel(E, N)

    m1, dis = pl.pallas_call(
        _pre_body,
        out_shape=(
            jax.ShapeDtypeStruct((H, N), jnp.float32),
            jax.ShapeDtypeStruct((1, N), jnp.float32),
        ),
    )(degp, x, W1.T)

    agg1 = agg_fn(m1.reshape(-1), packed).reshape(NC, H, N)

    mid = pl.pallas_call(
        _mid_body, out_shape=jax.ShapeDtypeStruct((H, N), jnp.float32)
    )

    m2 = mid(agg1, m1, dis, b1[:, None], g1[:, None], be1[:, None], W2.T)
    agg2 = agg_fn(m2.reshape(-1), packed).reshape(NC, H, N)

    m3 = mid(agg2, m2, dis, b2[:, None], g2[:, None], be2[:, None], W3.T)
    agg3 = agg_fn(m3.reshape(-1), packed).reshape(NC, H, N)

    return pl.pallas_call(
        _post_body, out_shape=jax.ShapeDtypeStruct((N, OUT), jnp.float32)
    )(agg3, m3, dis, b3[:, None], Wp1.T, bp1[:, None], Wp2, bp2[None, :])


# speedup vs baseline: 1.0266x; 1.0004x over previous
import functools
import jax
import jax.numpy as jnp
from jax import lax
from jax.experimental import pallas as pl
from jax.experimental.pallas import tpu as pltpu
from jax.experimental.pallas import tpu_sc as plsc

NC = 2
NS = 16
NW = NC * NS
L = 16

PACK_SHIFT = 14
PACK_MASK = (1 << PACK_SHIFT) - 1


def _wid():
    return lax.axis_index("s") * NC + lax.axis_index("c")


def _mesh():
    return plsc.VectorSubcoreMesh(
        core_axis_name="c", subcore_axis_name="s", num_cores=NC, num_subcores=NS
    )


def _deg_pack_kernel(E, N):
    Ew = E // NW
    assert E % (NW * L) == 0 and N % L == 0

    def body(edge_hbm, deg_hbm, packed_hbm, src_v, dst_v, pk_v, acc_v):
        w = _wid()
        base = w * Ew
        pltpu.sync_copy(edge_hbm.at[pl.ds(base, Ew)], src_v)
        pltpu.sync_copy(edge_hbm.at[pl.ds(E + base, Ew)], dst_v)

        def zero(i, c):
            acc_v[pl.ds(i * L, L)] = jnp.zeros((L,), jnp.float32)
            return c

        lax.fori_loop(0, N // L, zero, 0, unroll=8)

        ones = jnp.ones((L,), jnp.float32)

        @plsc.parallel_loop(0, Ew, step=L, unroll=8)
        def _(i):
            s16 = src_v[pl.ds(i, L)]
            d16 = dst_v[pl.ds(i, L)]
            pk_v[pl.ds(i, L)] = jnp.bitwise_or(
                jnp.left_shift(s16, PACK_SHIFT), d16
            )
            plsc.addupdate_scatter(acc_v, [d16], ones)
        pltpu.sync_copy(pk_v, packed_hbm.at[pl.ds(base, Ew)])
        pltpu.sync_copy(acc_v, deg_hbm.at[pl.ds(w * N, N)])

    return pl.kernel(
        body,
        out_type=(
            jax.ShapeDtypeStruct((NW * N,), jnp.float32),
            jax.ShapeDtypeStruct((E,), jnp.int32),
        ),
        mesh=_mesh(),
        compiler_params=pltpu.CompilerParams(needs_layout_passes=False),
        scratch_types=[
            pltpu.VMEM((Ew,), jnp.int32),
            pltpu.VMEM((Ew,), jnp.int32),
            pltpu.VMEM((Ew,), jnp.int32),
            pltpu.VMEM((N,), jnp.float32),
        ],
    )


def _agg_kernel(E, N, C=32000):
    Eh = E // NC
    while Eh % C or C % L:
        C //= 2
    nchunks = Eh // C
    assert Eh % C == 0 and C % L == 0 and N % L == 0

    def body(m_hbm, packed_hbm, out_hbm, m0, m1, a0, a1, pk0, pk1,
             sem0, sem1, semm):
        c = lax.axis_index("c")
        s = lax.axis_index("s")
        f0 = 2 * s
        ebase = c * Eh
        bufs = (pk0, pk1)
        sems = (sem0, sem1)
        mcp0 = pltpu.async_copy(m_hbm.at[pl.ds(f0 * N, N)], m0, semm)
        mcp1 = pltpu.async_copy(m_hbm.at[pl.ds((f0 + 1) * N, N)], m1, semm)
        handles = {0: pltpu.async_copy(packed_hbm.at[pl.ds(ebase, C)], pk0, sem0)}

        def zero(i, cc):
            a0[pl.ds(i * L, L)] = jnp.zeros((L,), jnp.float32)
            a1[pl.ds(i * L, L)] = jnp.zeros((L,), jnp.float32)
            return cc

        lax.fori_loop(0, N // L, zero, 0, unroll=8)
        mcp0.wait()
        mcp1.wait()

        for ci in range(nchunks):
            b = ci % 2
            if ci + 1 < nchunks:
                handles[ci + 1] = pltpu.async_copy(
                    packed_hbm.at[pl.ds(ebase + (ci + 1) * C, C)],
                    bufs[(ci + 1) % 2],
                    sems[(ci + 1) % 2],
                )
            handles[ci].wait()
            pk_v = bufs[b]

            @plsc.parallel_loop(0, C, step=L, unroll=16)
            def _(i):
                pk = pk_v[pl.ds(i, L)]
                s16 = lax.shift_right_logical(pk, PACK_SHIFT)
                d16 = jnp.bitwise_and(pk, PACK_MASK)
                plsc.addupdate_scatter(a0, [d16], plsc.load_gather(m0, [s16]))
                plsc.addupdate_scatter(a1, [d16], plsc.load_gather(m1, [s16]))

        obase = c * NW * N + f0 * N
        pltpu.sync_copy(a0, out_hbm.at[pl.ds(obase, N)])
        pltpu.sync_copy(a1, out_hbm.at[pl.ds(obase + N, N)])

    return pl.kernel(
        body,
        out_type=jax.ShapeDtypeStruct((NC * NW * N,), jnp.float32),
        mesh=_mesh(),
        compiler_params=pltpu.CompilerParams(needs_layout_passes=False),
        scratch_types=[
            pltpu.VMEM((N,), jnp.float32),
            pltpu.VMEM((N,), jnp.float32),
            pltpu.VMEM((N,), jnp.float32),
            pltpu.VMEM((N,), jnp.float32),
            pltpu.VMEM((C,), jnp.int32),
            pltpu.VMEM((C,), jnp.int32),
            pltpu.SemaphoreType.DMA,
            pltpu.SemaphoreType.DMA,
            pltpu.SemaphoreType.DMA,
        ],
    )


def _pre_body(degp_ref, x_ref, w1t_ref, m_ref, dis_ref):
    deg = jnp.sum(degp_ref[...], axis=0, keepdims=True) + 1.0
    dis = lax.rsqrt(deg)
    h = lax.dot_general(
        w1t_ref[...], x_ref[...], (((1,), (1,)), ((), ())),
        preferred_element_type=jnp.float32,
    )
    m_ref[...] = h * dis
    dis_ref[...] = dis


def _mid_body(agg_ref, m_ref, dis_ref, b_ref, g_ref, be_ref, wnt_ref, out_ref):
    dis = dis_ref[...]
    a = dis * (agg_ref[0] + agg_ref[1] + m_ref[...]) + b_ref[...]
    a = jnp.maximum(a, 0.0)
    mu = jnp.mean(a, axis=0, keepdims=True)
    var = jnp.mean((a - mu) * (a - mu), axis=0, keepdims=True)
    a = (a - mu) * lax.rsqrt(var + 1e-5) * g_ref[...] + be_ref[...]
    h = lax.dot_general(
        wnt_ref[...], a, (((1,), (0,)), ((), ())),
        preferred_element_type=jnp.float32,
    )
    out_ref[...] = h * dis


def _post_body(agg_ref, m_ref, dis_ref, b_ref, wp1t_ref, bp1_ref, wp2_ref,
               bp2_ref, out_ref):
    dis = dis_ref[...]
    a = dis * (agg_ref[0] + agg_ref[1] + m_ref[...]) + b_ref[...]
    a = jnp.maximum(a, 0.0)
    p = lax.dot_general(
        wp1t_ref[...], a, (((1,), (0,)), ((), ())),
        preferred_element_type=jnp.float32,
    ) + bp1_ref[...]
    p = jnp.maximum(p, 0.0)
    o = lax.dot_general(
        p, wp2_ref[...], (((0,), (0,)), ((), ())),
        preferred_element_type=jnp.float32,
    ) + bp2_ref[...]
    o = o - jnp.max(o, axis=1, keepdims=True)
    e = jnp.exp(o)
    out_ref[...] = e / jnp.sum(e, axis=1, keepdims=True)


@jax.jit
def kernel(x, edge_index, W1, b1, W2, b2, W3, b3, g1, be1, g2, be2,
           Wp1, bp1, Wp2, bp2):
    N, D = x.shape
    E = edge_index.shape[1]
    H = W1.shape[1]
    OUT = Wp2.shape[1]
    assert H == NW

    deg_flat, packed = _deg_pack_kernel(E, N)(edge_index.reshape(-1))
    degp = deg_flat.reshape(NW, N)

    agg_fn = _agg_kernel(E, N)

    m1, dis = pl.pallas_call(
        _pre_body,
        out_shape=(
            jax.ShapeDtypeStruct((H, N), jnp.float32),
            jax.ShapeDtypeStruct((1, N), jnp.float32),
        ),
    )(degp, x, W1.T)

    agg1 = agg_fn(m1.reshape(-1), packed).reshape(NC, H, N)

    mid = pl.pallas_call(
        _mid_body, out_shape=jax.ShapeDtypeStruct((H, N), jnp.float32)
    )

    m2 = mid(agg1, m1, dis, b1[:, None], g1[:, None], be1[:, None], W2.T)
    agg2 = agg_fn(m2.reshape(-1), packed).reshape(NC, H, N)

    m3 = mid(agg2, m2, dis, b2[:, None], g2[:, None], be2[:, None], W3.T)
    agg3 = agg_fn(m3.reshape(-1), packed).reshape(NC, H, N)

    return pl.pallas_call(
        _post_body, out_shape=jax.ShapeDtypeStruct((N, OUT), jnp.float32)
    )(agg3, m3, dis, b3[:, None], Wp1.T, bp1[:, None], Wp2, bp2[None, :])


# speedup vs baseline: 1.0278x; 1.0012x over previous
import jax
import jax.numpy as jnp
from jax import lax
from jax.experimental import pallas as pl
from jax.experimental.pallas import tpu as pltpu
from jax.experimental.pallas import tpu_sc as plsc

NC = 2
NS = 16
NW = NC * NS
L = 16

PACK_SHIFT = 14
PACK_MASK = (1 << PACK_SHIFT) - 1


def _wid():
    return lax.axis_index("s") * NC + lax.axis_index("c")


def _mesh():
    return plsc.VectorSubcoreMesh(
        core_axis_name="c", subcore_axis_name="s", num_cores=NC, num_subcores=NS
    )


def _deg_pack_kernel(E, N):
    Ew = E // NW
    assert E % (NW * L) == 0 and N % L == 0

    def body(edge_hbm, deg_hbm, packed_hbm, src_v, dst_v, pk_v, acc_v):
        w = _wid()
        base = w * Ew
        pltpu.sync_copy(edge_hbm.at[pl.ds(base, Ew)], src_v)
        pltpu.sync_copy(edge_hbm.at[pl.ds(E + base, Ew)], dst_v)

        def zero(i, c):
            acc_v[pl.ds(i * L, L)] = jnp.zeros((L,), jnp.float32)
            return c

        lax.fori_loop(0, N // L, zero, 0, unroll=8)

        ones = jnp.ones((L,), jnp.float32)

        @plsc.parallel_loop(0, Ew, step=L, unroll=8)
        def _(i):
            s16 = src_v[pl.ds(i, L)]
            d16 = dst_v[pl.ds(i, L)]
            pk_v[pl.ds(i, L)] = jnp.bitwise_or(
                jnp.left_shift(s16, PACK_SHIFT), d16
            )
            plsc.addupdate_scatter(acc_v, [d16], ones)
        pltpu.sync_copy(pk_v, packed_hbm.at[pl.ds(base, Ew)])
        pltpu.sync_copy(acc_v, deg_hbm.at[pl.ds(w * N, N)])

    return pl.kernel(
        body,
        out_type=(
            jax.ShapeDtypeStruct((NW * N,), jnp.float32),
            jax.ShapeDtypeStruct((E,), jnp.int32),
        ),
        mesh=_mesh(),
        compiler_params=pltpu.CompilerParams(needs_layout_passes=False),
        scratch_types=[
            pltpu.VMEM((Ew,), jnp.int32),
            pltpu.VMEM((Ew,), jnp.int32),
            pltpu.VMEM((Ew,), jnp.int32),
            pltpu.VMEM((N,), jnp.float32),
        ],
    )


def _agg_kernel(E, N, C=32000):
    Eh = E // NC
    while Eh % C or C % L:
        C //= 2
    nchunks = Eh // C
    assert Eh % C == 0 and C % L == 0 and N % L == 0

    def body(m_hbm, packed_hbm, out_hbm, m0, m1, a0, a1, pk0, pk1,
             sem0, sem1, semm):
        c = lax.axis_index("c")
        s = lax.axis_index("s")
        f0 = 2 * s
        ebase = c * Eh
        bufs = (pk0, pk1)
        sems = (sem0, sem1)
        mcp0 = pltpu.async_copy(m_hbm.at[pl.ds(f0 * N, N)], m0, semm)
        mcp1 = pltpu.async_copy(m_hbm.at[pl.ds((f0 + 1) * N, N)], m1, semm)
        handles = {0: pltpu.async_copy(packed_hbm.at[pl.ds(ebase, C)], pk0, sem0)}

        def zero(i, cc):
            a0[pl.ds(i * L, L)] = jnp.zeros((L,), jnp.float32)
            a1[pl.ds(i * L, L)] = jnp.zeros((L,), jnp.float32)
            return cc

        lax.fori_loop(0, N // L, zero, 0, unroll=8)
        mcp0.wait()
        mcp1.wait()

        for ci in range(nchunks):
            b = ci % 2
            if ci + 1 < nchunks:
                handles[ci + 1] = pltpu.async_copy(
                    packed_hbm.at[pl.ds(ebase + (ci + 1) * C, C)],
                    bufs[(ci + 1) % 2],
                    sems[(ci + 1) % 2],
                )
            handles[ci].wait()
            pk_v = bufs[b]

            @plsc.parallel_loop(0, C, step=L, unroll=16)
            def _(i):
                pk = pk_v[pl.ds(i, L)]
                s16 = lax.shift_right_logical(pk, PACK_SHIFT)
                d16 = jnp.bitwise_and(pk, PACK_MASK)
                plsc.addupdate_scatter(a0, [d16], plsc.load_gather(m0, [s16]))
                plsc.addupdate_scatter(a1, [d16], plsc.load_gather(m1, [s16]))

        obase = c * NW * N + f0 * N
        pltpu.sync_copy(a0, out_hbm.at[pl.ds(obase, N)])
        pltpu.sync_copy(a1, out_hbm.at[pl.ds(obase + N, N)])

    return pl.kernel(
        body,
        out_type=jax.ShapeDtypeStruct((NC * NW * N,), jnp.float32),
        mesh=_mesh(),
        compiler_params=pltpu.CompilerParams(needs_layout_passes=False),
        scratch_types=[
            pltpu.VMEM((N,), jnp.float32),
            pltpu.VMEM((N,), jnp.float32),
            pltpu.VMEM((N,), jnp.float32),
            pltpu.VMEM((N,), jnp.float32),
            pltpu.VMEM((C,), jnp.int32),
            pltpu.VMEM((C,), jnp.int32),
            pltpu.SemaphoreType.DMA,
            pltpu.SemaphoreType.DMA,
            pltpu.SemaphoreType.DMA,
        ],
    )


def _pre_body(degp_ref, x_ref, w1t_ref, m_ref, dis_ref):
    deg = jnp.sum(degp_ref[...], axis=0, keepdims=True) + 1.0
    dis = lax.rsqrt(deg)
    h = lax.dot_general(
        w1t_ref[...], x_ref[...], (((1,), (1,)), ((), ())),
        preferred_element_type=jnp.float32,
    )
    m_ref[...] = h * dis
    dis_ref[...] = dis


def _mid_body(agg_ref, m_ref, dis_ref, b_ref, g_ref, be_ref, wnt_ref, out_ref):
    dis = dis_ref[...]
    a = dis * (agg_ref[0] + agg_ref[1] + m_ref[...]) + b_ref[...]
    a = jnp.maximum(a, 0.0)
    mu = jnp.mean(a, axis=0, keepdims=True)
    var = jnp.mean((a - mu) * (a - mu), axis=0, keepdims=True)
    a = (a - mu) * lax.rsqrt(var + 1e-5) * g_ref[...] + be_ref[...]
    h = lax.dot_general(
        wnt_ref[...], a, (((1,), (0,)), ((), ())),
        preferred_element_type=jnp.float32,
    )
    out_ref[...] = h * dis


def _post_body(agg_ref, m_ref, dis_ref, b_ref, wp1t_ref, bp1_ref, wp2_ref,
               bp2_ref, out_ref):
    dis = dis_ref[...]
    a = dis * (agg_ref[0] + agg_ref[1] + m_ref[...]) + b_ref[...]
    a = jnp.maximum(a, 0.0)
    p = lax.dot_general(
        wp1t_ref[...], a, (((1,), (0,)), ((), ())),
        preferred_element_type=jnp.float32,
    ) + bp1_ref[...]
    p = jnp.maximum(p, 0.0)
    o = lax.dot_general(
        p, wp2_ref[...], (((0,), (0,)), ((), ())),
        preferred_element_type=jnp.float32,
    ) + bp2_ref[...]
    o = o - jnp.max(o, axis=1, keepdims=True)
    e = jnp.exp(o)
    out_ref[...] = e / jnp.sum(e, axis=1, keepdims=True)


@jax.jit
def kernel(x, edge_index, W1, b1, W2, b2, W3, b3, g1, be1, g2, be2,
           Wp1, bp1, Wp2, bp2):
    N, D = x.shape
    E = edge_index.shape[1]
    H = W1.shape[1]
    OUT = Wp2.shape[1]
    assert H == NW

    deg_flat, packed = _deg_pack_kernel(E, N)(edge_index.reshape(-1))
    degp = deg_flat.reshape(NW, N)

    agg_fn = _agg_kernel(E, N)

    m1, dis = pl.pallas_call(
        _pre_body,
        out_shape=(
            jax.ShapeDtypeStruct((H, N), jnp.float32),
            jax.ShapeDtypeStruct((1, N), jnp.float32),
        ),
    )(degp, x, W1.T)

    agg1 = agg_fn(m1.reshape(-1), packed).reshape(NC, H, N)

    mid = pl.pallas_call(
        _mid_body, out_shape=jax.ShapeDtypeStruct((H, N), jnp.float32)
    )

    m2 = mid(agg1, m1, dis, b1[:, None], g1[:, None], be1[:, None], W2.T)
    agg2 = agg_fn(m2.reshape(-1), packed).reshape(NC, H, N)

    m3 = mid(agg2, m2, dis, b2[:, None], g2[:, None], be2[:, None], W3.T)
    agg3 = agg_fn(m3.reshape(-1), packed).reshape(NC, H, N)

    return pl.pallas_call(
        _post_body, out_shape=jax.ShapeDtypeStruct((N, OUT), jnp.float32)
    )(agg3, m3, dis, b3[:, None], Wp1.T, bp1[:, None], Wp2, bp2[None, :])
